# baseline (speedup 1.0000x reference)
import jax
import jax.numpy as jnp
from jax import lax
from jax.experimental import pallas as pl
from jax.experimental.pallas import tpu as pltpu
from jax.experimental.pallas import tpu_sc as plsc

B, N, F = 1024, 200, 8
NUM_TYPES, EMBED_DIM = 8, 256

NC, NS, L = 2, 16, 16
NW = NC * NS
ROWS = B * N
PER_W = ROWS // NW
CHUNK = 128
N_CHUNKS = PER_W // CHUNK


def _sc_body(in_hbm, tab_hbm, out_hbm, in_v, idx_v, rows_v, sem):
    wid = lax.axis_index("s") * NC + lax.axis_index("c")
    base = wid * PER_W

    def chunk_body(c, carry):
        row0 = base + c * CHUNK
        pltpu.sync_copy(in_hbm.at[pl.ds(row0 * F, CHUNK * F)], in_v)
        strided = lax.iota(jnp.int32, L) * F + 2
        for j in range(CHUNK // L):
            vals = plsc.load_gather(in_v, [strided + (j * L * F)])
            idx = jnp.clip(vals.astype(jnp.int32), 0, NUM_TYPES - 1)
            idx_v[pl.ds(j * L, L)] = idx
        pltpu.async_copy(tab_hbm.at[idx_v], rows_v, sem).wait()
        pltpu.sync_copy(rows_v, out_hbm.at[pl.ds(row0, CHUNK)])
        return carry

    lax.fori_loop(0, N_CHUNKS, chunk_body, 0)


@jax.jit
def _sc_lookup(flat_inputs, type_embed):
    mesh = plsc.VectorSubcoreMesh(
        core_axis_name="c", subcore_axis_name="s",
        num_cores=NC, num_subcores=NS,
    )
    return pl.kernel(
        _sc_body,
        out_type=jax.ShapeDtypeStruct((ROWS, EMBED_DIM), jnp.float32),
        mesh=mesh,
        scratch_types=[
            pltpu.VMEM((CHUNK * F,), jnp.float32),
            pltpu.VMEM((CHUNK,), jnp.int32),
            pltpu.VMEM((CHUNK, EMBED_DIM), jnp.float32),
            pltpu.SemaphoreType.DMA,
        ],
        compiler_params=pltpu.CompilerParams(needs_layout_passes=False),
    )(flat_inputs, type_embed)


def kernel(inputs, type_embed):
    out = _sc_lookup(inputs.reshape(ROWS * F), type_embed)
    return out.reshape(B, N, EMBED_DIM)
